# baseline (speedup 1.0000x reference)
import functools

import jax
import jax.numpy as jnp
from jax import lax
from jax.experimental import pallas as pl
from jax.experimental.pallas import tpu as pltpu
from jax.experimental.pallas import tpu_sc as plsc

NUM_GRAPHS = 64

NC = 2
NS = 16
NW = NC * NS

CHUNK = 64


def _sc_edge_scatter(h_hbm, src_hbm, dst_hbm, z_hbm, p0_hbm, p1_hbm,
                     acc_sh, sidx_v, didx_v, rows_v, sems,
                     *, n_acc, n0, n1):
    cid = lax.axis_index("c")
    sid = lax.axis_index("s")

    rpt = n_acc // NS
    rz = pl.multiple_of(sid * rpt, 8)
    pltpu.sync_copy(z_hbm.at[pl.ds(rz, rpt)], acc_sh.at[pl.ds(rz, rpt)])
    plsc.subcore_barrier()

    nchunk = jnp.where(cid == 0, n0, n1)
    base = jnp.where(cid == 0, sid * n0, NS * n0 + sid * n1) * CHUNK

    def off(j):
        jc = jnp.minimum(j, nchunk - 1)
        return pl.multiple_of(base + jc * CHUNK, CHUNK)

    def gather(b):
        return pltpu.make_async_copy(h_hbm.at[sidx_v[b]], rows_v[b], sems[b])

    pltpu.sync_copy(src_hbm.at[pl.ds(off(0), CHUNK)], sidx_v[0])
    gather(0).start()

    def pair_body(i, _):
        j0 = 2 * i
        pltpu.sync_copy(src_hbm.at[pl.ds(off(j0 + 1), CHUNK)], sidx_v[1])
        gather(1).start()
        pltpu.sync_copy(dst_hbm.at[pl.ds(off(j0), CHUNK)], didx_v[0])
        gather(0).wait()
        pltpu.sync_copy(rows_v[0], acc_sh.at[didx_v[0]], add=True)
        pltpu.sync_copy(src_hbm.at[pl.ds(off(j0 + 2), CHUNK)], sidx_v[0])
        gather(0).start()
        pltpu.sync_copy(dst_hbm.at[pl.ds(off(j0 + 1), CHUNK)], didx_v[1])
        gather(1).wait()
        pltpu.sync_copy(rows_v[1], acc_sh.at[didx_v[1]], add=True)
        return 0

    lax.fori_loop(0, nchunk // 2, pair_body, 0)
    gather(0).wait()
    plsc.subcore_barrier()

    @pl.when(cid == 0)
    def _():
        pltpu.sync_copy(acc_sh.at[pl.ds(rz, rpt)], p0_hbm.at[pl.ds(rz, rpt)])

    @pl.when(cid == 1)
    def _():
        pltpu.sync_copy(acc_sh.at[pl.ds(rz, rpt)], p1_hbm.at[pl.ds(rz, rpt)])


def _tc_finish(h_ref, p0_ref, p1_ref, b_ref, w_ref, o_ref, *, n):
    out0 = h_ref[...] + p0_ref[0:n, :] + p1_ref[0:n, :]
    b = b_ref[...]
    gids = lax.broadcasted_iota(jnp.int32, (1, NUM_GRAPHS), 1)
    oh = (b == gids).astype(jnp.float32)
    dn_t = (((0,), (0,)), ((), ()))
    sums = lax.dot_general(oh, out0, dn_t,
                           preferred_element_type=jnp.float32)
    ones_col = jnp.ones_like(b, dtype=jnp.float32)
    counts = lax.dot_general(oh, ones_col, dn_t,
                             preferred_element_type=jnp.float32)
    vn = sums / jnp.maximum(counts, 1.0)
    dn = (((1,), (0,)), ((), ()))
    out1 = out0 + lax.dot_general(oh, vn, dn,
                                  preferred_element_type=jnp.float32)
    o_ref[...] = jnp.maximum(
        lax.dot_general(out1, w_ref[...], dn,
                        preferred_element_type=jnp.float32), 0.0)


def kernel(H, edge_index, batch, W):
    N, D = H.shape
    E = edge_index.shape[1]

    total = 2 * ((E + NS * CHUNK * 2 - 1) // (NS * CHUNK * 2))
    n0 = 2 * int(round(total * 0.30 / 2))
    n1 = total - n0
    e_pad = NS * total * CHUNK
    n_acc = ((N + 1 + NS * 8 - 1) // (NS * 8)) * (NS * 8)
    pad = e_pad - E
    src = jnp.concatenate(
        [edge_index[0], jnp.zeros((pad,), jnp.int32)]).astype(jnp.int32)
    dst = jnp.concatenate(
        [edge_index[1], jnp.full((pad,), N, jnp.int32)]).astype(jnp.int32)
    zeros = jnp.zeros((n_acc, D), jnp.float32)

    mesh = plsc.VectorSubcoreMesh(core_axis_name="c", subcore_axis_name="s")
    sc_fn = pl.kernel(
        functools.partial(_sc_edge_scatter, n_acc=n_acc, n0=n0, n1=n1),
        out_type=(
            jax.ShapeDtypeStruct((n_acc, D), jnp.float32),
            jax.ShapeDtypeStruct((n_acc, D), jnp.float32),
        ),
        mesh=mesh,
        scratch_types=[
            pltpu.VMEM_SHARED((n_acc, D), jnp.float32),
            [pltpu.VMEM((CHUNK,), jnp.int32) for _ in range(2)],
            [pltpu.VMEM((CHUNK,), jnp.int32) for _ in range(2)],
            [pltpu.VMEM((CHUNK, D), jnp.float32) for _ in range(2)],
            [pltpu.SemaphoreType.DMA for _ in range(2)],
        ],
    )
    p0, p1 = sc_fn(H, src, dst, zeros)

    out = pl.pallas_call(
        functools.partial(_tc_finish, n=N),
        out_shape=jax.ShapeDtypeStruct((N, D), jnp.float32),
    )(H, p0, p1, batch.astype(jnp.int32).reshape(N, 1), W)
    return out


# speedup vs baseline: 1.1393x; 1.1393x over previous
import functools

import jax
import jax.numpy as jnp
from jax import lax
from jax.experimental import pallas as pl
from jax.experimental.pallas import tpu as pltpu
from jax.experimental.pallas import tpu_sc as plsc

NUM_GRAPHS = 64

NC = 2
NS = 16
NW = NC * NS

CHUNK = 64


def _sc_edge_scatter(h_hbm, src_hbm, dst_hbm, z_hbm, p0_hbm, p1_hbm,
                     acc_sh, sidx_v, didx_v, rows_v, sems,
                     *, n_acc, n0, n1):
    cid = lax.axis_index("c")
    sid = lax.axis_index("s")

    rpt = n_acc // NS
    rz = pl.multiple_of(sid * rpt, 8)
    pltpu.sync_copy(z_hbm.at[pl.ds(rz, rpt)], acc_sh.at[pl.ds(rz, rpt)])
    plsc.subcore_barrier()

    nchunk = jnp.where(cid == 0, n0, n1)
    base = jnp.where(cid == 0, sid * n0, NS * n0 + sid * n1) * CHUNK

    def off(j):
        jc = jnp.minimum(j, nchunk - 1)
        return pl.multiple_of(base + jc * CHUNK, CHUNK)

    def gather(b):
        return pltpu.make_async_copy(h_hbm.at[sidx_v[b]], rows_v[b], sems[b])

    pltpu.sync_copy(src_hbm.at[pl.ds(off(0), CHUNK)], sidx_v[0])
    gather(0).start()

    def pair_body(i, _):
        j0 = 2 * i
        pltpu.sync_copy(src_hbm.at[pl.ds(off(j0 + 1), CHUNK)], sidx_v[1])
        gather(1).start()
        pltpu.sync_copy(dst_hbm.at[pl.ds(off(j0), CHUNK)], didx_v[0])
        gather(0).wait()
        pltpu.sync_copy(rows_v[0], acc_sh.at[didx_v[0]], add=True)
        pltpu.sync_copy(src_hbm.at[pl.ds(off(j0 + 2), CHUNK)], sidx_v[0])
        gather(0).start()
        pltpu.sync_copy(dst_hbm.at[pl.ds(off(j0 + 1), CHUNK)], didx_v[1])
        gather(1).wait()
        pltpu.sync_copy(rows_v[1], acc_sh.at[didx_v[1]], add=True)
        return 0

    lax.fori_loop(0, nchunk // 2, pair_body, 0)
    gather(0).wait()
    plsc.subcore_barrier()

    @pl.when(cid == 0)
    def _():
        pltpu.sync_copy(acc_sh.at[pl.ds(rz, rpt)], p0_hbm.at[pl.ds(rz, rpt)])

    @pl.when(cid == 1)
    def _():
        pltpu.sync_copy(acc_sh.at[pl.ds(rz, rpt)], p1_hbm.at[pl.ds(rz, rpt)])


def _tc_finish(h_ref, p0_ref, p1_ref, b_ref, w_ref, o_ref, *, n):
    out0 = h_ref[...] + p0_ref[0:n, :] + p1_ref[0:n, :]
    b = b_ref[...]
    gids = lax.broadcasted_iota(jnp.int32, (1, NUM_GRAPHS), 1)
    oh = (b == gids).astype(jnp.float32)
    dn_t = (((0,), (0,)), ((), ()))
    sums = lax.dot_general(oh, out0, dn_t,
                           preferred_element_type=jnp.float32)
    ones_col = jnp.ones_like(b, dtype=jnp.float32)
    counts = lax.dot_general(oh, ones_col, dn_t,
                             preferred_element_type=jnp.float32)
    vn = sums / jnp.maximum(counts, 1.0)
    dn = (((1,), (0,)), ((), ()))
    out1 = out0 + lax.dot_general(oh, vn, dn,
                                  preferred_element_type=jnp.float32)
    o_ref[...] = jnp.maximum(
        lax.dot_general(out1, w_ref[...], dn,
                        preferred_element_type=jnp.float32), 0.0)


def kernel(H, edge_index, batch, W):
    N, D = H.shape
    E = edge_index.shape[1]

    total = 2 * ((E + NS * CHUNK * 2 - 1) // (NS * CHUNK * 2))
    n0 = 2 * int(round(total * 0.42 / 2))
    n1 = total - n0
    e_pad = NS * total * CHUNK
    n_acc = ((N + 1 + NS * 8 - 1) // (NS * 8)) * (NS * 8)
    pad = e_pad - E
    src = jnp.concatenate(
        [edge_index[0], jnp.zeros((pad,), jnp.int32)]).astype(jnp.int32)
    dst = jnp.concatenate(
        [edge_index[1], jnp.full((pad,), N, jnp.int32)]).astype(jnp.int32)
    zeros = jnp.zeros((n_acc, D), jnp.float32)

    mesh = plsc.VectorSubcoreMesh(core_axis_name="c", subcore_axis_name="s")
    sc_fn = pl.kernel(
        functools.partial(_sc_edge_scatter, n_acc=n_acc, n0=n0, n1=n1),
        out_type=(
            jax.ShapeDtypeStruct((n_acc, D), jnp.float32),
            jax.ShapeDtypeStruct((n_acc, D), jnp.float32),
        ),
        mesh=mesh,
        scratch_types=[
            pltpu.VMEM_SHARED((n_acc, D), jnp.float32),
            [pltpu.VMEM((CHUNK,), jnp.int32) for _ in range(2)],
            [pltpu.VMEM((CHUNK,), jnp.int32) for _ in range(2)],
            [pltpu.VMEM((CHUNK, D), jnp.float32) for _ in range(2)],
            [pltpu.SemaphoreType.DMA for _ in range(2)],
        ],
    )
    p0, p1 = sc_fn(H, src, dst, zeros)

    out = pl.pallas_call(
        functools.partial(_tc_finish, n=N),
        out_shape=jax.ShapeDtypeStruct((N, D), jnp.float32),
    )(H, p0, p1, batch.astype(jnp.int32).reshape(N, 1), W)
    return out
